# baseline (speedup 1.0000x reference)
import dataclasses
import functools

import jax
import jax.numpy as jnp
from jax import lax
from jax.experimental import pallas as pl
from jax.experimental.pallas import tpu as pltpu
from jax.experimental.pallas import tpu_sc as plsc

N = 10000
NFEAT = 128
NHID = 128
NCLASS = 64

NCORES = 2
NSUB = 16
NW = NCORES * NSUB
CH = 128
N_PAD = 10240
NCH0 = 136

f32 = jnp.float32
i32 = jnp.int32


def _sc_degree(col, w, n_pad):
    e_pad = col.shape[0]
    ept = e_pad // NW
    nchunks = ept // CH

    mesh = plsc.VectorSubcoreMesh(core_axis_name="c", subcore_axis_name="s")

    cp = pltpu.CompilerParams()
    if "needs_layout_passes" in pltpu.CompilerParams.__dataclass_fields__:
        cp = dataclasses.replace(cp, needs_layout_passes=False)

    @functools.partial(
        pl.kernel, mesh=mesh,
        out_type=jax.ShapeDtypeStruct((NW, n_pad), f32),
        compiler_params=cp,
        scratch_types=[
            pltpu.VMEM((CH,), i32),
            pltpu.VMEM((CH,), f32),
            pltpu.VMEM((n_pad,), f32),
        ],
    )
    def k(col_hbm, w_hbm, out_hbm, cbuf, wbuf, pdeg):
        cid = lax.axis_index("c")
        sid = lax.axis_index("s")
        wid = sid * NCORES + cid

        zero16 = jnp.zeros((16,), f32)

        @pl.loop(0, n_pad, step=16)
        def _(j):
            pdeg.at[pl.ds(j, 16)][...] = zero16

        base0 = wid * ept

        @pl.loop(0, nchunks)
        def _(ci):
            base = base0 + ci * CH
            pltpu.sync_copy(col_hbm.at[pl.ds(base, CH)], cbuf)
            pltpu.sync_copy(w_hbm.at[pl.ds(base, CH)], wbuf)

            @pl.loop(0, CH, step=16)
            def _(g):
                cv = cbuf[pl.ds(g, 16)]
                wv = wbuf[pl.ds(g, 16)]
                plsc.addupdate_scatter(pdeg, [cv], wv)

        pltpu.sync_copy(pdeg, out_hbm.at[wid])

    return k(col, w)


def _sc_propagate(xp, row2, col2, w2, n_pad, feat):
    nchunks_total = row2.shape[0]
    nchunks = nchunks_total // NW
    rps = n_pad // NSUB

    mesh = plsc.VectorSubcoreMesh(core_axis_name="c", subcore_axis_name="s")

    idx_t = [pltpu.VMEM((CH,), i32)] * 2
    w_t = [pltpu.VMEM((CH,), f32)] * 2
    rows_t = [pltpu.VMEM((CH, feat), f32)] * 2

    @functools.partial(
        pl.kernel, mesh=mesh,
        out_type=jax.ShapeDtypeStruct((NCORES, n_pad, feat), f32),
        scratch_types=idx_t + idx_t + w_t + rows_t + [
            pltpu.VMEM_SHARED((n_pad, feat), f32),
            pltpu.SemaphoreType.DMA,
            pltpu.SemaphoreType.DMA,
            pltpu.SemaphoreType.DMA,
            pltpu.SemaphoreType.DMA,
        ],
    )
    def k(x_hbm, row_hbm, col_hbm, w_hbm, out_hbm,
          rb0, rb1, cb0, cb1, wb0, wb1, rows0, rows1, acc,
          sg0, sg1, si0, si1):
        cid = lax.axis_index("c")
        sid = lax.axis_index("s")
        wid = sid * NCORES + cid

        rb = (rb0, rb1)
        cb = (cb0, cb1)
        wb = (wb0, wb1)
        rows = (rows0, rows1)
        sg = (sg0, sg1)
        si = (si0, si1)

        zero16 = jnp.zeros((16,), f32)

        @pl.loop(0, CH)
        def _(i):
            @pl.loop(0, feat, step=16)
            def _(j):
                rows0.at[i, pl.ds(j, 16)][...] = zero16

        @pl.loop(0, rps, step=CH)
        def _(r):
            pltpu.sync_copy(rows0, acc.at[pl.ds(sid * rps + r, CH)])

        plsc.subcore_barrier()

        nch_c = jnp.where(cid == 0, NCH0, 2 * nchunks - NCH0)
        cbase = sid * 2 * nchunks + jnp.where(cid == 0, 0, NCH0)
        last = nch_c - 1

        def issue_idx(b, c):
            pltpu.async_copy(row_hbm.at[c + cbase], rb[b], si[b])
            pltpu.async_copy(col_hbm.at[c + cbase], cb[b], si[b])
            pltpu.async_copy(w_hbm.at[c + cbase], wb[b], si[b])

        def wait_idx(b, c):
            pltpu.make_async_copy(row_hbm.at[c + cbase], rb[b], si[b]).wait()
            pltpu.make_async_copy(col_hbm.at[c + cbase], cb[b], si[b]).wait()
            pltpu.make_async_copy(w_hbm.at[c + cbase], wb[b], si[b]).wait()

        def scale(b):
            rw = rows[b]
            wbb = wb[b]

            @pl.loop(0, CH, step=16)
            def _(g):
                wv = wbb[pl.ds(g, 16)]
                for l in range(16):
                    we = wv[l]
                    for j in range(0, feat, 16):
                        rw.at[g + l, pl.ds(j, 16)][...] = (
                            rw.at[g + l, pl.ds(j, 16)][...] * we)

        def half(b, c, cnxt1, cnxt2):
            wait_idx(1 - b, cnxt1)
            pltpu.async_copy(x_hbm.at[rb[1 - b]], rows[1 - b], sg[1 - b])
            pltpu.make_async_copy(x_hbm.at[rb[b]], rows[b], sg[b]).wait()
            scale(b)
            pltpu.sync_copy(rows[b], acc.at[cb[b]], add=True)
            issue_idx(b, cnxt2)

        issue_idx(0, 0)
        wait_idx(0, 0)
        pltpu.async_copy(x_hbm.at[rb[0]], rows0, sg[0])
        issue_idx(1, 1)

        @pl.loop(0, nch_c, step=2)
        def _(ci):
            half(0, ci, ci + 1, jnp.minimum(ci + 2, last))
            half(1, ci + 1, jnp.minimum(ci + 2, last),
                 jnp.minimum(ci + 3, last))

        pltpu.make_async_copy(x_hbm.at[rb[0]], rows0, sg[0]).wait()
        wait_idx(1, last)

        plsc.subcore_barrier()

        @pl.loop(0, rps, step=CH)
        def _(r):
            pltpu.sync_copy(acc.at[pl.ds(sid * rps + r, CH)],
                            out_hbm.at[cid, pl.ds(sid * rps + r, CH)])

    return k(xp, row2, col2, w2)


BR = 1024
DIS_BLOCK = (BR, 1)


def _tc_dis(degw):
    def body(degw_ref, dis_ref):
        ones = jnp.ones((NW, 1), f32)
        deg = lax.dot_general(degw_ref[...], ones, (((0,), (0,)), ((), ())),
                              precision=lax.Precision.HIGHEST,
                              preferred_element_type=f32) + 1.0
        dis_ref[...] = jnp.where(
            deg > 0, lax.rsqrt(jnp.maximum(deg, 1e-12)), 0.0)

    return pl.pallas_call(
        body,
        grid=(N_PAD // BR,),
        in_specs=[pl.BlockSpec((NW, BR), lambda i: (0, i))],
        out_specs=pl.BlockSpec((BR, 1), lambda i: (i, 0)),
        out_shape=jax.ShapeDtypeStruct((N_PAD, 1), f32),
    )(degw)


def _tc_prescale(dis_img, x_pad):
    def body(dis_ref, x_ref, o_ref):
        o_ref[...] = dis_ref[...] * x_ref[...]

    return pl.pallas_call(
        body,
        grid=(N_PAD // BR,),
        in_specs=[
            pl.BlockSpec(DIS_BLOCK, lambda i: (i, 0)),
            pl.BlockSpec((BR, NFEAT), lambda i: (i, 0)),
        ],
        out_specs=pl.BlockSpec((BR, NFEAT), lambda i: (i, 0)),
        out_shape=jax.ShapeDtypeStruct((N_PAD, NFEAT), f32),
    )(dis_img, x_pad)


def _tc_mid(dis_img, s1, xp, W1, b1, W2):
    def body(dis_ref, s1_ref, xp_ref, w1_ref, b1_ref, w2_ref, o_ref):
        dis = dis_ref[...]
        t = s1_ref[0] + s1_ref[1] + xp_ref[...]
        z = dis * t
        h = lax.dot_general(z, w1_ref[...], (((1,), (1,)), ((), ())),
                            precision=lax.Precision.HIGHEST,
                            preferred_element_type=f32)
        h = jnp.maximum(h + b1_ref[...], 0.0)
        g = lax.dot_general(h, w2_ref[...], (((1,), (1,)), ((), ())),
                            precision=lax.Precision.HIGHEST,
                            preferred_element_type=f32)
        o_ref[...] = jnp.concatenate(
            [dis * g, jnp.zeros((BR, NHID - NCLASS), f32)], axis=1)

    return pl.pallas_call(
        body,
        grid=(N_PAD // BR,),
        in_specs=[
            pl.BlockSpec(DIS_BLOCK, lambda i: (i, 0)),
            pl.BlockSpec((NCORES, BR, NHID), lambda i: (0, i, 0)),
            pl.BlockSpec((BR, NFEAT), lambda i: (i, 0)),
            pl.BlockSpec((NHID, NFEAT), lambda i: (0, 0)),
            pl.BlockSpec((1, NHID), lambda i: (0, 0)),
            pl.BlockSpec((NCLASS, NHID), lambda i: (0, 0)),
        ],
        out_specs=pl.BlockSpec((BR, NHID), lambda i: (i, 0)),
        out_shape=jax.ShapeDtypeStruct((N_PAD, NHID), f32),
    )(dis_img, s1, xp, W1, b1, W2)


def _tc_final(dis_img, s2, gp, b2):
    def body(dis_ref, s2_ref, gp_ref, b2_ref, o_ref):
        dis = dis_ref[...]
        t = (s2_ref[0] + s2_ref[1] + gp_ref[...])[:, :NCLASS]
        z = dis * t + b2_ref[...]
        m = jnp.max(z, axis=1, keepdims=True)
        zm = z - m
        s = jnp.sum(jnp.exp(zm), axis=1, keepdims=True)
        o_ref[...] = zm - jnp.log(s)

    return pl.pallas_call(
        body,
        grid=(N_PAD // BR,),
        in_specs=[
            pl.BlockSpec(DIS_BLOCK, lambda i: (i, 0)),
            pl.BlockSpec((NCORES, BR, NHID), lambda i: (0, i, 0)),
            pl.BlockSpec((BR, NHID), lambda i: (i, 0)),
            pl.BlockSpec((1, NCLASS), lambda i: (0, 0)),
        ],
        out_specs=pl.BlockSpec((BR, NCLASS), lambda i: (i, 0)),
        out_shape=jax.ShapeDtypeStruct((N_PAD, NCLASS), f32),
    )(dis_img, s2, gp, b2)


@jax.jit
def _run(features, edge_index, edge_weight, W1, b1, W2, b2):
    E = edge_index.shape[1]
    chunk = NW * CH * 2
    e_pad = ((E + chunk - 1) // chunk) * chunk
    pad = e_pad - E

    row = jnp.pad(edge_index[0], (0, pad))
    col = jnp.pad(edge_index[1], (0, pad))
    w = jnp.pad(edge_weight, (0, pad))
    row2 = row.reshape(-1, CH)
    col2 = col.reshape(-1, CH)
    w2 = w.reshape(-1, CH)

    x_pad = jnp.pad(features, ((0, N_PAD - N), (0, 0)))

    degw = _sc_degree(col, w, N_PAD)
    dis_img = _tc_dis(degw)
    xp = _tc_prescale(dis_img, x_pad)
    s1 = _sc_propagate(xp, row2, col2, w2, N_PAD, NFEAT)
    gp = _tc_mid(dis_img, s1, xp, W1, b1.reshape(1, NHID), W2)
    s2 = _sc_propagate(gp, row2, col2, w2, N_PAD, NHID)
    out = _tc_final(dis_img, s2, gp, b2.reshape(1, NCLASS))
    return out[:N]


def kernel(features, edge_index, edge_weight, W1, b1, W2, b2):
    return _run(features, edge_index, edge_weight, W1, b1, W2, b2)


# speedup vs baseline: 1.0393x; 1.0393x over previous
import dataclasses
import functools

import jax
import jax.numpy as jnp
from jax import lax
from jax.experimental import pallas as pl
from jax.experimental.pallas import tpu as pltpu
from jax.experimental.pallas import tpu_sc as plsc

N = 10000
NFEAT = 128
NHID = 128
NCLASS = 64

NCORES = 2
NSUB = 16
NW = NCORES * NSUB
CH = 128
N_PAD = 10240
NCH0 = 128

f32 = jnp.float32
i32 = jnp.int32


def _sc_degree(col, w, n_pad):
    e_pad = col.shape[0]
    ept = e_pad // NW
    nchunks = ept // CH

    mesh = plsc.VectorSubcoreMesh(core_axis_name="c", subcore_axis_name="s")

    cp = pltpu.CompilerParams()
    if "needs_layout_passes" in pltpu.CompilerParams.__dataclass_fields__:
        cp = dataclasses.replace(cp, needs_layout_passes=False)

    @functools.partial(
        pl.kernel, mesh=mesh,
        out_type=jax.ShapeDtypeStruct((NW, n_pad), f32),
        compiler_params=cp,
        scratch_types=[
            pltpu.VMEM((CH,), i32),
            pltpu.VMEM((CH,), f32),
            pltpu.VMEM((n_pad,), f32),
        ],
    )
    def k(col_hbm, w_hbm, out_hbm, cbuf, wbuf, pdeg):
        cid = lax.axis_index("c")
        sid = lax.axis_index("s")
        wid = sid * NCORES + cid

        zero16 = jnp.zeros((16,), f32)

        @pl.loop(0, n_pad, step=16)
        def _(j):
            pdeg.at[pl.ds(j, 16)][...] = zero16

        base0 = wid * ept

        @pl.loop(0, nchunks)
        def _(ci):
            base = base0 + ci * CH
            pltpu.sync_copy(col_hbm.at[pl.ds(base, CH)], cbuf)
            pltpu.sync_copy(w_hbm.at[pl.ds(base, CH)], wbuf)

            @pl.loop(0, CH, step=16)
            def _(g):
                cv = cbuf[pl.ds(g, 16)]
                wv = wbuf[pl.ds(g, 16)]
                plsc.addupdate_scatter(pdeg, [cv], wv)

        pltpu.sync_copy(pdeg, out_hbm.at[wid])

    return k(col, w)


def _sc_propagate(xp, row2, col2, w2, n_pad, feat):
    nchunks_total = row2.shape[0]
    nchunks = nchunks_total // NW
    rps = n_pad // NSUB

    mesh = plsc.VectorSubcoreMesh(core_axis_name="c", subcore_axis_name="s")

    idx_t = [pltpu.VMEM((CH,), i32)] * 2
    w_t = [pltpu.VMEM((CH,), f32)] * 2
    rows_t = [pltpu.VMEM((CH, feat), f32)] * 2

    @functools.partial(
        pl.kernel, mesh=mesh,
        out_type=jax.ShapeDtypeStruct((NCORES, n_pad, feat), f32),
        scratch_types=idx_t + idx_t + w_t + rows_t + [
            pltpu.VMEM_SHARED((n_pad, feat), f32),
            pltpu.SemaphoreType.DMA,
            pltpu.SemaphoreType.DMA,
            pltpu.SemaphoreType.DMA,
            pltpu.SemaphoreType.DMA,
        ],
    )
    def k(x_hbm, row_hbm, col_hbm, w_hbm, out_hbm,
          rb0, rb1, cb0, cb1, wb0, wb1, rows0, rows1, acc,
          sg0, sg1, si0, si1):
        cid = lax.axis_index("c")
        sid = lax.axis_index("s")
        wid = sid * NCORES + cid

        rb = (rb0, rb1)
        cb = (cb0, cb1)
        wb = (wb0, wb1)
        rows = (rows0, rows1)
        sg = (sg0, sg1)
        si = (si0, si1)

        zero16 = jnp.zeros((16,), f32)

        @pl.loop(0, CH)
        def _(i):
            @pl.loop(0, feat, step=16)
            def _(j):
                rows0.at[i, pl.ds(j, 16)][...] = zero16

        @pl.loop(0, rps, step=CH)
        def _(r):
            pltpu.sync_copy(rows0, acc.at[pl.ds(sid * rps + r, CH)])

        plsc.subcore_barrier()

        nch_c = jnp.where(cid == 0, NCH0, 2 * nchunks - NCH0)
        cbase = sid * 2 * nchunks + jnp.where(cid == 0, 0, NCH0)
        last = nch_c - 1

        def issue_idx(b, c):
            pltpu.async_copy(row_hbm.at[c + cbase], rb[b], si[b])
            pltpu.async_copy(col_hbm.at[c + cbase], cb[b], si[b])
            pltpu.async_copy(w_hbm.at[c + cbase], wb[b], si[b])

        def wait_idx(b, c):
            pltpu.make_async_copy(row_hbm.at[c + cbase], rb[b], si[b]).wait()
            pltpu.make_async_copy(col_hbm.at[c + cbase], cb[b], si[b]).wait()
            pltpu.make_async_copy(w_hbm.at[c + cbase], wb[b], si[b]).wait()

        def scale(b):
            rw = rows[b]
            wbb = wb[b]

            @pl.loop(0, CH, step=16)
            def _(g):
                wv = wbb[pl.ds(g, 16)]
                for l in range(16):
                    we = wv[l]
                    for j in range(0, feat, 16):
                        rw.at[g + l, pl.ds(j, 16)][...] = (
                            rw.at[g + l, pl.ds(j, 16)][...] * we)

        def half(b, c, cnxt1, cnxt2):
            wait_idx(1 - b, cnxt1)
            pltpu.async_copy(x_hbm.at[rb[1 - b]], rows[1 - b], sg[1 - b])
            pltpu.make_async_copy(x_hbm.at[rb[b]], rows[b], sg[b]).wait()
            scale(b)
            pltpu.sync_copy(rows[b], acc.at[cb[b]], add=True)
            issue_idx(b, cnxt2)

        issue_idx(0, 0)
        wait_idx(0, 0)
        pltpu.async_copy(x_hbm.at[rb[0]], rows0, sg[0])
        issue_idx(1, 1)

        @pl.loop(0, nch_c, step=2)
        def _(ci):
            half(0, ci, ci + 1, jnp.minimum(ci + 2, last))
            half(1, ci + 1, jnp.minimum(ci + 2, last),
                 jnp.minimum(ci + 3, last))

        pltpu.make_async_copy(x_hbm.at[rb[0]], rows0, sg[0]).wait()
        wait_idx(1, last)

        plsc.subcore_barrier()

        @pl.loop(0, rps, step=CH)
        def _(r):
            pltpu.sync_copy(acc.at[pl.ds(sid * rps + r, CH)],
                            out_hbm.at[cid, pl.ds(sid * rps + r, CH)])

    return k(xp, row2, col2, w2)


BR = 1024
DIS_BLOCK = (BR, 1)


def _tc_dis(degw):
    def body(degw_ref, dis_ref):
        ones = jnp.ones((NW, 1), f32)
        deg = lax.dot_general(degw_ref[...], ones, (((0,), (0,)), ((), ())),
                              precision=lax.Precision.HIGHEST,
                              preferred_element_type=f32) + 1.0
        dis_ref[...] = jnp.where(
            deg > 0, lax.rsqrt(jnp.maximum(deg, 1e-12)), 0.0)

    return pl.pallas_call(
        body,
        grid=(N_PAD // BR,),
        in_specs=[pl.BlockSpec((NW, BR), lambda i: (0, i))],
        out_specs=pl.BlockSpec((BR, 1), lambda i: (i, 0)),
        out_shape=jax.ShapeDtypeStruct((N_PAD, 1), f32),
    )(degw)


def _tc_prescale(dis_img, x_pad):
    def body(dis_ref, x_ref, o_ref):
        o_ref[...] = dis_ref[...] * x_ref[...]

    return pl.pallas_call(
        body,
        grid=(N_PAD // BR,),
        in_specs=[
            pl.BlockSpec(DIS_BLOCK, lambda i: (i, 0)),
            pl.BlockSpec((BR, NFEAT), lambda i: (i, 0)),
        ],
        out_specs=pl.BlockSpec((BR, NFEAT), lambda i: (i, 0)),
        out_shape=jax.ShapeDtypeStruct((N_PAD, NFEAT), f32),
    )(dis_img, x_pad)


def _tc_mid(dis_img, s1, xp, W1, b1, W2):
    def body(dis_ref, s1_ref, xp_ref, w1_ref, b1_ref, w2_ref, o_ref):
        dis = dis_ref[...]
        t = s1_ref[0] + s1_ref[1] + xp_ref[...]
        z = dis * t
        h = lax.dot_general(z, w1_ref[...], (((1,), (1,)), ((), ())),
                            precision=lax.Precision.HIGHEST,
                            preferred_element_type=f32)
        h = jnp.maximum(h + b1_ref[...], 0.0)
        g = lax.dot_general(h, w2_ref[...], (((1,), (1,)), ((), ())),
                            precision=lax.Precision.HIGHEST,
                            preferred_element_type=f32)
        o_ref[...] = jnp.concatenate(
            [dis * g, jnp.zeros((BR, NHID - NCLASS), f32)], axis=1)

    return pl.pallas_call(
        body,
        grid=(N_PAD // BR,),
        in_specs=[
            pl.BlockSpec(DIS_BLOCK, lambda i: (i, 0)),
            pl.BlockSpec((NCORES, BR, NHID), lambda i: (0, i, 0)),
            pl.BlockSpec((BR, NFEAT), lambda i: (i, 0)),
            pl.BlockSpec((NHID, NFEAT), lambda i: (0, 0)),
            pl.BlockSpec((1, NHID), lambda i: (0, 0)),
            pl.BlockSpec((NCLASS, NHID), lambda i: (0, 0)),
        ],
        out_specs=pl.BlockSpec((BR, NHID), lambda i: (i, 0)),
        out_shape=jax.ShapeDtypeStruct((N_PAD, NHID), f32),
    )(dis_img, s1, xp, W1, b1, W2)


def _tc_final(dis_img, s2, gp, b2):
    def body(dis_ref, s2_ref, gp_ref, b2_ref, o_ref):
        dis = dis_ref[...]
        t = (s2_ref[0] + s2_ref[1] + gp_ref[...])[:, :NCLASS]
        z = dis * t + b2_ref[...]
        m = jnp.max(z, axis=1, keepdims=True)
        zm = z - m
        s = jnp.sum(jnp.exp(zm), axis=1, keepdims=True)
        o_ref[...] = zm - jnp.log(s)

    return pl.pallas_call(
        body,
        grid=(N_PAD // BR,),
        in_specs=[
            pl.BlockSpec(DIS_BLOCK, lambda i: (i, 0)),
            pl.BlockSpec((NCORES, BR, NHID), lambda i: (0, i, 0)),
            pl.BlockSpec((BR, NHID), lambda i: (i, 0)),
            pl.BlockSpec((1, NCLASS), lambda i: (0, 0)),
        ],
        out_specs=pl.BlockSpec((BR, NCLASS), lambda i: (i, 0)),
        out_shape=jax.ShapeDtypeStruct((N_PAD, NCLASS), f32),
    )(dis_img, s2, gp, b2)


@jax.jit
def _run(features, edge_index, edge_weight, W1, b1, W2, b2):
    E = edge_index.shape[1]
    chunk = NW * CH * 2
    e_pad = ((E + chunk - 1) // chunk) * chunk
    pad = e_pad - E

    row = jnp.pad(edge_index[0], (0, pad))
    col = jnp.pad(edge_index[1], (0, pad))
    w = jnp.pad(edge_weight, (0, pad))
    row2 = row.reshape(-1, CH)
    col2 = col.reshape(-1, CH)
    w2 = w.reshape(-1, CH)

    x_pad = jnp.pad(features, ((0, N_PAD - N), (0, 0)))

    degw = _sc_degree(col, w, N_PAD)
    dis_img = _tc_dis(degw)
    xp = _tc_prescale(dis_img, x_pad)
    s1 = _sc_propagate(xp, row2, col2, w2, N_PAD, NFEAT)
    gp = _tc_mid(dis_img, s1, xp, W1, b1.reshape(1, NHID), W2)
    s2 = _sc_propagate(gp, row2, col2, w2, N_PAD, NHID)
    out = _tc_final(dis_img, s2, gp, b2.reshape(1, NCLASS))
    return out[:N]


def kernel(features, edge_index, edge_weight, W1, b1, W2, b2):
    return _run(features, edge_index, edge_weight, W1, b1, W2, b2)


# speedup vs baseline: 1.0938x; 1.0525x over previous
import dataclasses
import functools

import jax
import jax.numpy as jnp
from jax import lax
from jax.experimental import pallas as pl
from jax.experimental.pallas import tpu as pltpu
from jax.experimental.pallas import tpu_sc as plsc

N = 10000
NFEAT = 128
NHID = 128
NCLASS = 64

NCORES = 2
NSUB = 16
NW = NCORES * NSUB
CH = 128
N_PAD = 10240
NCH0 = 112

f32 = jnp.float32
i32 = jnp.int32


def _sc_degree(col, w, n_pad):
    e_pad = col.shape[0]
    ept = e_pad // NW
    nchunks = ept // CH

    mesh = plsc.VectorSubcoreMesh(core_axis_name="c", subcore_axis_name="s")

    cp = pltpu.CompilerParams()
    if "needs_layout_passes" in pltpu.CompilerParams.__dataclass_fields__:
        cp = dataclasses.replace(cp, needs_layout_passes=False)

    @functools.partial(
        pl.kernel, mesh=mesh,
        out_type=jax.ShapeDtypeStruct((NW, n_pad), f32),
        compiler_params=cp,
        scratch_types=[
            pltpu.VMEM((CH,), i32),
            pltpu.VMEM((CH,), f32),
            pltpu.VMEM((n_pad,), f32),
        ],
    )
    def k(col_hbm, w_hbm, out_hbm, cbuf, wbuf, pdeg):
        cid = lax.axis_index("c")
        sid = lax.axis_index("s")
        wid = sid * NCORES + cid

        zero16 = jnp.zeros((16,), f32)

        @pl.loop(0, n_pad, step=16)
        def _(j):
            pdeg.at[pl.ds(j, 16)][...] = zero16

        base0 = wid * ept

        @pl.loop(0, nchunks)
        def _(ci):
            base = base0 + ci * CH
            pltpu.sync_copy(col_hbm.at[pl.ds(base, CH)], cbuf)
            pltpu.sync_copy(w_hbm.at[pl.ds(base, CH)], wbuf)

            @pl.loop(0, CH, step=16)
            def _(g):
                cv = cbuf[pl.ds(g, 16)]
                wv = wbuf[pl.ds(g, 16)]
                plsc.addupdate_scatter(pdeg, [cv], wv)

        pltpu.sync_copy(pdeg, out_hbm.at[wid])

    return k(col, w)


def _sc_propagate(xp, row2, col2, w2, n_pad, feat):
    nchunks_total = row2.shape[0]
    nchunks = nchunks_total // NW
    rps = n_pad // NSUB

    mesh = plsc.VectorSubcoreMesh(core_axis_name="c", subcore_axis_name="s")

    idx_t = [pltpu.VMEM((CH,), i32)] * 2
    w_t = [pltpu.VMEM((CH,), f32)] * 2
    rows_t = [pltpu.VMEM((CH, feat), f32)] * 2

    @functools.partial(
        pl.kernel, mesh=mesh,
        out_type=jax.ShapeDtypeStruct((NCORES, n_pad, feat), f32),
        scratch_types=idx_t + idx_t + w_t + rows_t + [
            pltpu.VMEM_SHARED((n_pad, feat), f32),
            pltpu.SemaphoreType.DMA,
            pltpu.SemaphoreType.DMA,
            pltpu.SemaphoreType.DMA,
            pltpu.SemaphoreType.DMA,
        ],
    )
    def k(x_hbm, row_hbm, col_hbm, w_hbm, out_hbm,
          rb0, rb1, cb0, cb1, wb0, wb1, rows0, rows1, acc,
          sg0, sg1, si0, si1):
        cid = lax.axis_index("c")
        sid = lax.axis_index("s")
        wid = sid * NCORES + cid

        rb = (rb0, rb1)
        cb = (cb0, cb1)
        wb = (wb0, wb1)
        rows = (rows0, rows1)
        sg = (sg0, sg1)
        si = (si0, si1)

        zero16 = jnp.zeros((16,), f32)

        @pl.loop(0, CH)
        def _(i):
            @pl.loop(0, feat, step=16)
            def _(j):
                rows0.at[i, pl.ds(j, 16)][...] = zero16

        @pl.loop(0, rps, step=CH)
        def _(r):
            pltpu.sync_copy(rows0, acc.at[pl.ds(sid * rps + r, CH)])

        plsc.subcore_barrier()

        nch_c = jnp.where(cid == 0, NCH0, 2 * nchunks - NCH0)
        cbase = sid * 2 * nchunks + jnp.where(cid == 0, 0, NCH0)
        last = nch_c - 1

        def issue_idx(b, c):
            pltpu.async_copy(row_hbm.at[c + cbase], rb[b], si[b])
            pltpu.async_copy(col_hbm.at[c + cbase], cb[b], si[b])
            pltpu.async_copy(w_hbm.at[c + cbase], wb[b], si[b])

        def wait_idx(b, c):
            pltpu.make_async_copy(row_hbm.at[c + cbase], rb[b], si[b]).wait()
            pltpu.make_async_copy(col_hbm.at[c + cbase], cb[b], si[b]).wait()
            pltpu.make_async_copy(w_hbm.at[c + cbase], wb[b], si[b]).wait()

        def scale(b):
            rw = rows[b]
            wbb = wb[b]

            @pl.loop(0, CH, step=16)
            def _(g):
                wv = wbb[pl.ds(g, 16)]
                for l in range(16):
                    we = wv[l]
                    for j in range(0, feat, 16):
                        rw.at[g + l, pl.ds(j, 16)][...] = (
                            rw.at[g + l, pl.ds(j, 16)][...] * we)

        def half(b, c, cnxt1, cnxt2):
            wait_idx(1 - b, cnxt1)
            pltpu.async_copy(x_hbm.at[rb[1 - b]], rows[1 - b], sg[1 - b])
            pltpu.make_async_copy(x_hbm.at[rb[b]], rows[b], sg[b]).wait()
            scale(b)
            pltpu.sync_copy(rows[b], acc.at[cb[b]], add=True)
            issue_idx(b, cnxt2)

        issue_idx(0, 0)
        wait_idx(0, 0)
        pltpu.async_copy(x_hbm.at[rb[0]], rows0, sg[0])
        issue_idx(1, 1)

        @pl.loop(0, nch_c, step=2)
        def _(ci):
            half(0, ci, ci + 1, jnp.minimum(ci + 2, last))
            half(1, ci + 1, jnp.minimum(ci + 2, last),
                 jnp.minimum(ci + 3, last))

        pltpu.make_async_copy(x_hbm.at[rb[0]], rows0, sg[0]).wait()
        wait_idx(1, last)

        plsc.subcore_barrier()

        @pl.loop(0, rps, step=CH)
        def _(r):
            pltpu.sync_copy(acc.at[pl.ds(sid * rps + r, CH)],
                            out_hbm.at[cid, pl.ds(sid * rps + r, CH)])

    return k(xp, row2, col2, w2)


BR = 1024
DIS_BLOCK = (BR, 1)


def _tc_dis(degw):
    def body(degw_ref, dis_ref):
        ones = jnp.ones((NW, 1), f32)
        deg = lax.dot_general(degw_ref[...], ones, (((0,), (0,)), ((), ())),
                              precision=lax.Precision.HIGHEST,
                              preferred_element_type=f32) + 1.0
        dis_ref[...] = jnp.where(
            deg > 0, lax.rsqrt(jnp.maximum(deg, 1e-12)), 0.0)

    return pl.pallas_call(
        body,
        grid=(N_PAD // BR,),
        in_specs=[pl.BlockSpec((NW, BR), lambda i: (0, i))],
        out_specs=pl.BlockSpec((BR, 1), lambda i: (i, 0)),
        out_shape=jax.ShapeDtypeStruct((N_PAD, 1), f32),
    )(degw)


def _tc_prescale(dis_img, x_pad):
    def body(dis_ref, x_ref, o_ref):
        o_ref[...] = dis_ref[...] * x_ref[...]

    return pl.pallas_call(
        body,
        grid=(N_PAD // BR,),
        in_specs=[
            pl.BlockSpec(DIS_BLOCK, lambda i: (i, 0)),
            pl.BlockSpec((BR, NFEAT), lambda i: (i, 0)),
        ],
        out_specs=pl.BlockSpec((BR, NFEAT), lambda i: (i, 0)),
        out_shape=jax.ShapeDtypeStruct((N_PAD, NFEAT), f32),
    )(dis_img, x_pad)


def _tc_mid(dis_img, s1, xp, W1, b1, W2):
    def body(dis_ref, s1_ref, xp_ref, w1_ref, b1_ref, w2_ref, o_ref):
        dis = dis_ref[...]
        t = s1_ref[0] + s1_ref[1] + xp_ref[...]
        z = dis * t
        h = lax.dot_general(z, w1_ref[...], (((1,), (1,)), ((), ())),
                            precision=lax.Precision.HIGHEST,
                            preferred_element_type=f32)
        h = jnp.maximum(h + b1_ref[...], 0.0)
        g = lax.dot_general(h, w2_ref[...], (((1,), (1,)), ((), ())),
                            precision=lax.Precision.HIGHEST,
                            preferred_element_type=f32)
        o_ref[...] = jnp.concatenate(
            [dis * g, jnp.zeros((BR, NHID - NCLASS), f32)], axis=1)

    return pl.pallas_call(
        body,
        grid=(N_PAD // BR,),
        in_specs=[
            pl.BlockSpec(DIS_BLOCK, lambda i: (i, 0)),
            pl.BlockSpec((NCORES, BR, NHID), lambda i: (0, i, 0)),
            pl.BlockSpec((BR, NFEAT), lambda i: (i, 0)),
            pl.BlockSpec((NHID, NFEAT), lambda i: (0, 0)),
            pl.BlockSpec((1, NHID), lambda i: (0, 0)),
            pl.BlockSpec((NCLASS, NHID), lambda i: (0, 0)),
        ],
        out_specs=pl.BlockSpec((BR, NHID), lambda i: (i, 0)),
        out_shape=jax.ShapeDtypeStruct((N_PAD, NHID), f32),
    )(dis_img, s1, xp, W1, b1, W2)


def _tc_final(dis_img, s2, gp, b2):
    def body(dis_ref, s2_ref, gp_ref, b2_ref, o_ref):
        dis = dis_ref[...]
        t = (s2_ref[0] + s2_ref[1] + gp_ref[...])[:, :NCLASS]
        z = dis * t + b2_ref[...]
        m = jnp.max(z, axis=1, keepdims=True)
        zm = z - m
        s = jnp.sum(jnp.exp(zm), axis=1, keepdims=True)
        o_ref[...] = zm - jnp.log(s)

    return pl.pallas_call(
        body,
        grid=(N_PAD // BR,),
        in_specs=[
            pl.BlockSpec(DIS_BLOCK, lambda i: (i, 0)),
            pl.BlockSpec((NCORES, BR, NHID), lambda i: (0, i, 0)),
            pl.BlockSpec((BR, NHID), lambda i: (i, 0)),
            pl.BlockSpec((1, NCLASS), lambda i: (0, 0)),
        ],
        out_specs=pl.BlockSpec((BR, NCLASS), lambda i: (i, 0)),
        out_shape=jax.ShapeDtypeStruct((N_PAD, NCLASS), f32),
    )(dis_img, s2, gp, b2)


@jax.jit
def _run(features, edge_index, edge_weight, W1, b1, W2, b2):
    E = edge_index.shape[1]
    chunk = NW * CH * 2
    e_pad = ((E + chunk - 1) // chunk) * chunk
    pad = e_pad - E

    row = jnp.pad(edge_index[0], (0, pad))
    col = jnp.pad(edge_index[1], (0, pad))
    w = jnp.pad(edge_weight, (0, pad))
    row2 = row.reshape(-1, CH)
    col2 = col.reshape(-1, CH)
    w2 = w.reshape(-1, CH)

    x_pad = jnp.pad(features, ((0, N_PAD - N), (0, 0)))

    degw = _sc_degree(col, w, N_PAD)
    dis_img = _tc_dis(degw)
    xp = _tc_prescale(dis_img, x_pad)
    s1 = _sc_propagate(xp, row2, col2, w2, N_PAD, NFEAT)
    gp = _tc_mid(dis_img, s1, xp, W1, b1.reshape(1, NHID), W2)
    s2 = _sc_propagate(gp, row2, col2, w2, N_PAD, NHID)
    out = _tc_final(dis_img, s2, gp, b2.reshape(1, NCLASS))
    return out[:N]


def kernel(features, edge_index, edge_weight, W1, b1, W2, b2):
    return _run(features, edge_index, edge_weight, W1, b1, W2, b2)


# speedup vs baseline: 1.1826x; 1.0812x over previous
import dataclasses
import functools

import jax
import jax.numpy as jnp
from jax import lax
from jax.experimental import pallas as pl
from jax.experimental.pallas import tpu as pltpu
from jax.experimental.pallas import tpu_sc as plsc

N = 10000
NFEAT = 128
NHID = 128
NCLASS = 64

NCORES = 2
NSUB = 16
NW = NCORES * NSUB
CH = 128
N_PAD = 10240
NCH0 = 120

f32 = jnp.float32
i32 = jnp.int32


def _sc_degree(col, w, n_pad):
    e_pad = col.shape[0]
    ept = e_pad // NW
    E_PT = ept

    mesh = plsc.VectorSubcoreMesh(core_axis_name="c", subcore_axis_name="s")

    cp = pltpu.CompilerParams()
    if "needs_layout_passes" in pltpu.CompilerParams.__dataclass_fields__:
        cp = dataclasses.replace(cp, needs_layout_passes=False)

    @functools.partial(
        pl.kernel, mesh=mesh,
        out_type=jax.ShapeDtypeStruct((NW, n_pad), f32),
        compiler_params=cp,
        scratch_types=[
            pltpu.VMEM((E_PT,), i32),
            pltpu.VMEM((E_PT,), f32),
            pltpu.VMEM((n_pad,), f32),
            pltpu.SemaphoreType.DMA,
        ],
    )
    def k(col_hbm, w_hbm, out_hbm, cbuf, wbuf, pdeg, sem):
        cid = lax.axis_index("c")
        sid = lax.axis_index("s")
        wid = sid * NCORES + cid

        base0 = wid * ept
        pltpu.async_copy(col_hbm.at[pl.ds(base0, ept)], cbuf, sem)
        pltpu.async_copy(w_hbm.at[pl.ds(base0, ept)], wbuf, sem)

        zero16 = jnp.zeros((16,), f32)

        @pl.loop(0, n_pad, step=16)
        def _(j):
            pdeg.at[pl.ds(j, 16)][...] = zero16

        pltpu.make_async_copy(col_hbm.at[pl.ds(base0, ept)], cbuf, sem).wait()
        pltpu.make_async_copy(w_hbm.at[pl.ds(base0, ept)], wbuf, sem).wait()

        @pl.loop(0, ept, step=16)
        def _(g):
            cv = cbuf[pl.ds(g, 16)]
            wv = wbuf[pl.ds(g, 16)]
            plsc.addupdate_scatter(pdeg, [cv], wv)

        pltpu.sync_copy(pdeg, out_hbm.at[wid])

    return k(col, w)


def _sc_propagate(xp, row2, col2, w2, n_pad, feat):
    nchunks_total = row2.shape[0]
    nchunks = nchunks_total // NW
    rps = n_pad // NSUB

    mesh = plsc.VectorSubcoreMesh(core_axis_name="c", subcore_axis_name="s")

    idx_t = [pltpu.VMEM((CH,), i32)] * 2
    w_t = [pltpu.VMEM((CH,), f32)] * 2
    rows_t = [pltpu.VMEM((CH, feat), f32)] * 2

    @functools.partial(
        pl.kernel, mesh=mesh,
        out_type=jax.ShapeDtypeStruct((NCORES, n_pad, feat), f32),
        scratch_types=idx_t + idx_t + w_t + rows_t + [
            pltpu.VMEM_SHARED((n_pad, feat), f32),
            pltpu.SemaphoreType.DMA,
            pltpu.SemaphoreType.DMA,
            pltpu.SemaphoreType.DMA,
            pltpu.SemaphoreType.DMA,
        ],
    )
    def k(x_hbm, row_hbm, col_hbm, w_hbm, out_hbm,
          rb0, rb1, cb0, cb1, wb0, wb1, rows0, rows1, acc,
          sg0, sg1, si0, si1):
        cid = lax.axis_index("c")
        sid = lax.axis_index("s")
        wid = sid * NCORES + cid

        rb = (rb0, rb1)
        cb = (cb0, cb1)
        wb = (wb0, wb1)
        rows = (rows0, rows1)
        sg = (sg0, sg1)
        si = (si0, si1)

        zero16 = jnp.zeros((16,), f32)

        @pl.loop(0, CH)
        def _(i):
            @pl.loop(0, feat, step=16)
            def _(j):
                rows0.at[i, pl.ds(j, 16)][...] = zero16

        @pl.loop(0, rps, step=CH)
        def _(r):
            pltpu.sync_copy(rows0, acc.at[pl.ds(sid * rps + r, CH)])

        plsc.subcore_barrier()

        nch_c = jnp.where(cid == 0, NCH0, 2 * nchunks - NCH0)
        cbase = sid * 2 * nchunks + jnp.where(cid == 0, 0, NCH0)
        last = nch_c - 1

        def issue_idx(b, c):
            pltpu.async_copy(row_hbm.at[c + cbase], rb[b], si[b])
            pltpu.async_copy(col_hbm.at[c + cbase], cb[b], si[b])
            pltpu.async_copy(w_hbm.at[c + cbase], wb[b], si[b])

        def wait_idx(b, c):
            pltpu.make_async_copy(row_hbm.at[c + cbase], rb[b], si[b]).wait()
            pltpu.make_async_copy(col_hbm.at[c + cbase], cb[b], si[b]).wait()
            pltpu.make_async_copy(w_hbm.at[c + cbase], wb[b], si[b]).wait()

        def scale(b):
            rw = rows[b]
            wbb = wb[b]

            @pl.loop(0, CH, step=16)
            def _(g):
                wv = wbb[pl.ds(g, 16)]
                for l in range(16):
                    we = wv[l]
                    for j in range(0, feat, 16):
                        rw.at[g + l, pl.ds(j, 16)][...] = (
                            rw.at[g + l, pl.ds(j, 16)][...] * we)

        def half(b, c, cnxt1, cnxt2):
            wait_idx(1 - b, cnxt1)
            pltpu.async_copy(x_hbm.at[rb[1 - b]], rows[1 - b], sg[1 - b])
            pltpu.make_async_copy(x_hbm.at[rb[b]], rows[b], sg[b]).wait()
            scale(b)
            pltpu.sync_copy(rows[b], acc.at[cb[b]], add=True)
            issue_idx(b, cnxt2)

        issue_idx(0, 0)
        wait_idx(0, 0)
        pltpu.async_copy(x_hbm.at[rb[0]], rows0, sg[0])
        issue_idx(1, 1)

        @pl.loop(0, nch_c, step=2)
        def _(ci):
            half(0, ci, ci + 1, jnp.minimum(ci + 2, last))
            half(1, ci + 1, jnp.minimum(ci + 2, last),
                 jnp.minimum(ci + 3, last))

        pltpu.make_async_copy(x_hbm.at[rb[0]], rows0, sg[0]).wait()
        wait_idx(1, last)

        plsc.subcore_barrier()

        @pl.loop(0, rps, step=CH)
        def _(r):
            pltpu.sync_copy(acc.at[pl.ds(sid * rps + r, CH)],
                            out_hbm.at[cid, pl.ds(sid * rps + r, CH)])

    return k(xp, row2, col2, w2)


BR = 1024
DIS_BLOCK = (BR, 1)


def _tc_dis(degw):
    def body(degw_ref, dis_ref):
        ones = jnp.ones((NW, 1), f32)
        deg = lax.dot_general(degw_ref[...], ones, (((0,), (0,)), ((), ())),
                              precision=lax.Precision.HIGHEST,
                              preferred_element_type=f32) + 1.0
        dis_ref[...] = jnp.where(
            deg > 0, lax.rsqrt(jnp.maximum(deg, 1e-12)), 0.0)

    return pl.pallas_call(
        body,
        grid=(N_PAD // BR,),
        in_specs=[pl.BlockSpec((NW, BR), lambda i: (0, i))],
        out_specs=pl.BlockSpec((BR, 1), lambda i: (i, 0)),
        out_shape=jax.ShapeDtypeStruct((N_PAD, 1), f32),
    )(degw)


def _tc_prescale(dis_img, x_pad):
    def body(dis_ref, x_ref, o_ref):
        o_ref[...] = dis_ref[...] * x_ref[...]

    return pl.pallas_call(
        body,
        grid=(N_PAD // BR,),
        in_specs=[
            pl.BlockSpec(DIS_BLOCK, lambda i: (i, 0)),
            pl.BlockSpec((BR, NFEAT), lambda i: (i, 0)),
        ],
        out_specs=pl.BlockSpec((BR, NFEAT), lambda i: (i, 0)),
        out_shape=jax.ShapeDtypeStruct((N_PAD, NFEAT), f32),
    )(dis_img, x_pad)


def _tc_mid(dis_img, s1, xp, W1, b1, W2):
    def body(dis_ref, s1_ref, xp_ref, w1_ref, b1_ref, w2_ref, o_ref):
        dis = dis_ref[...]
        t = s1_ref[0] + s1_ref[1] + xp_ref[...]
        z = dis * t
        h = lax.dot_general(z, w1_ref[...], (((1,), (1,)), ((), ())),
                            precision=lax.Precision.HIGHEST,
                            preferred_element_type=f32)
        h = jnp.maximum(h + b1_ref[...], 0.0)
        g = lax.dot_general(h, w2_ref[...], (((1,), (1,)), ((), ())),
                            precision=lax.Precision.HIGHEST,
                            preferred_element_type=f32)
        o_ref[...] = jnp.concatenate(
            [dis * g, jnp.zeros((BR, NHID - NCLASS), f32)], axis=1)

    return pl.pallas_call(
        body,
        grid=(N_PAD // BR,),
        in_specs=[
            pl.BlockSpec(DIS_BLOCK, lambda i: (i, 0)),
            pl.BlockSpec((NCORES, BR, NHID), lambda i: (0, i, 0)),
            pl.BlockSpec((BR, NFEAT), lambda i: (i, 0)),
            pl.BlockSpec((NHID, NFEAT), lambda i: (0, 0)),
            pl.BlockSpec((1, NHID), lambda i: (0, 0)),
            pl.BlockSpec((NCLASS, NHID), lambda i: (0, 0)),
        ],
        out_specs=pl.BlockSpec((BR, NHID), lambda i: (i, 0)),
        out_shape=jax.ShapeDtypeStruct((N_PAD, NHID), f32),
    )(dis_img, s1, xp, W1, b1, W2)


def _tc_final(dis_img, s2, gp, b2):
    def body(dis_ref, s2_ref, gp_ref, b2_ref, o_ref):
        dis = dis_ref[...]
        t = (s2_ref[0] + s2_ref[1] + gp_ref[...])[:, :NCLASS]
        z = dis * t + b2_ref[...]
        m = jnp.max(z, axis=1, keepdims=True)
        zm = z - m
        s = jnp.sum(jnp.exp(zm), axis=1, keepdims=True)
        o_ref[...] = zm - jnp.log(s)

    return pl.pallas_call(
        body,
        grid=(N_PAD // BR,),
        in_specs=[
            pl.BlockSpec(DIS_BLOCK, lambda i: (i, 0)),
            pl.BlockSpec((NCORES, BR, NHID), lambda i: (0, i, 0)),
            pl.BlockSpec((BR, NHID), lambda i: (i, 0)),
            pl.BlockSpec((1, NCLASS), lambda i: (0, 0)),
        ],
        out_specs=pl.BlockSpec((BR, NCLASS), lambda i: (i, 0)),
        out_shape=jax.ShapeDtypeStruct((N_PAD, NCLASS), f32),
    )(dis_img, s2, gp, b2)


@jax.jit
def _run(features, edge_index, edge_weight, W1, b1, W2, b2):
    E = edge_index.shape[1]
    chunk = NW * CH * 2
    e_pad = ((E + chunk - 1) // chunk) * chunk
    pad = e_pad - E

    row = jnp.pad(edge_index[0], (0, pad))
    col = jnp.pad(edge_index[1], (0, pad))
    w = jnp.pad(edge_weight, (0, pad))
    row2 = row.reshape(-1, CH)
    col2 = col.reshape(-1, CH)
    w2 = w.reshape(-1, CH)

    x_pad = jnp.pad(features, ((0, N_PAD - N), (0, 0)))

    degw = _sc_degree(col, w, N_PAD)
    dis_img = _tc_dis(degw)
    xp = _tc_prescale(dis_img, x_pad)
    s1 = _sc_propagate(xp, row2, col2, w2, N_PAD, NFEAT)
    gp = _tc_mid(dis_img, s1, xp, W1, b1.reshape(1, NHID), W2)
    s2 = _sc_propagate(gp, row2, col2, w2, N_PAD, NHID)
    out = _tc_final(dis_img, s2, gp, b2.reshape(1, NCLASS))
    return out[:N]


def kernel(features, edge_index, edge_weight, W1, b1, W2, b2):
    return _run(features, edge_index, edge_weight, W1, b1, W2, b2)
